# baseline (speedup 1.0000x reference)
import functools

import jax
import jax.numpy as jnp
from jax import lax
from jax.experimental import pallas as pl
from jax.experimental.pallas import tpu as pltpu
from jax.experimental.pallas import tpu_sc as plsc

B, CDD, HIS, K = 16, 5, 100, 5
SIG, LVL, HID = 32, 3, 256
THRESHOLD = 0.1
ROW = SIG * LVL * HID
NROWS = B * CDD * K
NC, NS = 2, 16
NW = NC * NS
CHUNKS = ROW // 16
RPW = -(-NROWS // NW)


def _select_body(cdd_ref, his_ref, wt_ref, b_ref, gidx_ref, vals_ref):
    bidx = pl.program_id(0)
    cdd = cdd_ref[0]
    his = his_ref[0]
    wt = wt_ref[...]
    bias = b_ref[...]
    dn = (((1,), (0,)), ((), ()))
    cp = lax.dot_general(cdd, wt, dn, preferred_element_type=jnp.float32) + bias
    hp = lax.dot_general(his, wt, dn, preferred_element_type=jnp.float32) + bias
    cn = jnp.sqrt(jnp.sum(cp * cp, axis=1, keepdims=True))
    hn = jnp.sqrt(jnp.sum(hp * hp, axis=1, keepdims=True))
    cp = cp / jnp.maximum(cn, 1e-12)
    hp = hp / jnp.maximum(hn, 1e-12)
    attn = lax.dot_general(cp, hp, (((1,), (1,)), ((), ())),
                           preferred_element_type=jnp.float32)
    cols = lax.broadcasted_iota(jnp.int32, (CDD, HIS), 1)
    a = attn
    vs, ids = [], []
    for _ in range(K):
        m = jnp.max(a, axis=1, keepdims=True)
        amax = jnp.min(jnp.where(a == m, cols, HIS), axis=1, keepdims=True)
        vs.append(m)
        ids.append(amax)
        a = jnp.where(cols == amax, -jnp.inf, a)
    vals = jnp.concatenate(vs, axis=1)
    idx = jnp.concatenate(ids, axis=1)
    masked = jnp.where(vals < THRESHOLD, 0.0, vals)
    gidx_ref[0] = idx + bidx * HIS
    vals_ref[0] = masked


def _select(cdd_repr, his_repr, sel_Wt, sel_b2):
    return pl.pallas_call(
        _select_body,
        grid=(B,),
        in_specs=[
            pl.BlockSpec((1, CDD, HID), lambda b: (b, 0, 0)),
            pl.BlockSpec((1, HIS, HID), lambda b: (b, 0, 0)),
            pl.BlockSpec((HID, HID), lambda b: (0, 0)),
            pl.BlockSpec((1, HID), lambda b: (0, 0)),
        ],
        out_specs=[
            pl.BlockSpec((1, CDD, K), lambda b: (b, 0, 0)),
            pl.BlockSpec((1, CDD, K), lambda b: (b, 0, 0)),
        ],
        out_shape=[
            jax.ShapeDtypeStruct((B, CDD, K), jnp.int32),
            jax.ShapeDtypeStruct((B, CDD, K), jnp.float32),
        ],
    )(cdd_repr, his_repr, sel_Wt, sel_b2)


def _gather_body(table, gspread, vrep, out, idx_v, wbuf, rowbuf, sem):
    c = lax.axis_index("c")
    s = lax.axis_index("s")
    wid = s * NC + c
    pltpu.sync_copy(gspread.at[wid], idx_v)

    def row_body(i, carry):
        row = wid + i * NW

        @pl.when(row < NROWS)
        def _():
            pltpu.sync_copy(vrep.at[pl.ds(row, 1)], wbuf)
            off = pl.multiple_of(i * 8, 8)
            pltpu.async_copy(table.at[idx_v.at[pl.ds(off, 1)]],
                             rowbuf, sem).wait()
            wvec = wbuf[0]

            def mul(j, carry2):
                sl = pl.ds(j * 16, 16)
                rowbuf[0, sl] = rowbuf[0, sl] * wvec
                return carry2
            lax.fori_loop(0, CHUNKS, mul, 0, unroll=8)
            pltpu.sync_copy(rowbuf, out.at[pl.ds(row, 1)])
        return carry
    lax.fori_loop(0, RPW, row_body, 0)


@functools.cache
def _gather():
    return pl.kernel(
        _gather_body,
        mesh=plsc.VectorSubcoreMesh(core_axis_name="c", subcore_axis_name="s"),
        out_type=jax.ShapeDtypeStruct((NROWS, ROW), jnp.float32),
        scratch_types=[
            pltpu.VMEM((RPW * 8,), jnp.int32),
            pltpu.VMEM((1, 16), jnp.float32),
            pltpu.VMEM((1, ROW), jnp.float32),
            pltpu.SemaphoreType.DMA,
        ],
    )


def kernel(cdd_repr, his_repr, his_embedding, sel_W, sel_b):
    gidx, vals = _select(cdd_repr, his_repr, sel_W.T, sel_b.reshape(1, HID))
    table = his_embedding.reshape(B * HIS, ROW)
    vrep = jnp.broadcast_to(vals.reshape(NROWS, 1), (NROWS, 16))
    gpad = jnp.pad(gidx.reshape(NROWS), (0, NW * RPW - NROWS))
    gmat = gpad.reshape(RPW, NW).T
    gspread = jnp.pad(gmat[:, :, None],
                      ((0, 0), (0, 0), (0, 7))).reshape(NW, RPW * 8)
    out = _gather()(table, gspread, vrep)
    return out.reshape(B, CDD, K, SIG, LVL, HID)
